# baseline (speedup 1.0000x reference)
import functools

import jax
import jax.numpy as jnp
from jax import lax
from jax.experimental import pallas as pl
from jax.experimental.pallas import tpu as pltpu
from jax.experimental.pallas import tpu_sc as plsc

N = 10000
E = 160000
D = 256

NP = 10240
NB = 1000
EB = 2000
K = 40
EPT = E // 16
NCHUNK = EPT // K
RPT = NP // 16
RQ = 16
RK = RPT // RQ


def _block_mm(x_ref, w_ref, o_ref):
    o_ref[0] = jnp.dot(x_ref[...], w_ref[0], preferred_element_type=jnp.float32)


def _node_kernel(x_ref, wam_ref, wbr_ref, bias_ref, am_ref, br_ref):
    xb = x_ref[...]
    am_ref[0] = jnp.dot(xb, wam_ref[0], preferred_element_type=jnp.float32)
    br = jnp.dot(xb, wbr_ref[0], preferred_element_type=jnp.float32) + bias_ref[0]
    j = pl.program_id(0)

    @pl.when(j == 2)
    def _():
        br_ref[0] = br + xb[:, :128]

    @pl.when(j == 3)
    def _():
        br_ref[0] = br + xb[:, 128:]

    @pl.when(j < 2)
    def _():
        br_ref[0] = br


def _final_kernel(acc_ref, r_ref, o_ref):
    a0 = acc_ref[0]
    a1 = acc_ref[1]
    msg = jnp.concatenate([a0[:, :64], a1[:, :64]], axis=1)
    norm = jnp.concatenate([a0[:, 64:], a1[:, 64:]], axis=1)
    o_ref[...] = msg / jnp.maximum(norm, 1e-6) + r_ref[0]


def _sc_edge_kernel(am_hbm, b2_hbm, g2_hbm, row4_hbm, col2_hbm, col_hbm,
                    acc_hbm, acc_sp, rowi_a, colgi_a, colsi_a, rowi_b,
                    colgi_b, colsi_b, am_a, b_a, g_a, mg_a,
                    am_b, b_b, g_b, mg_b, sem_ia, sem_ib, sem_sa, sem_sb,
                    sem_g, sem_s):
    c = lax.axis_index("c")
    s = lax.axis_index("s")
    coff = c * 64
    rbase = s * RPT
    ebase = s * EPT

    def compute_edges(am_buf, b_buf, g_buf, mg_buf, nedges):
        @plsc.parallel_loop(0, nedges, step=1, unroll=4)
        def edge_body(e):
            for g in range(4):
                a = am_buf[e, pl.ds(g * 16, 16)]
                m = am_buf[e, pl.ds(64 + g * 16, 16)]
                bv = b_buf[e, pl.ds(coff + g * 16, 16)]
                gv = g_buf[e, pl.ds(coff + g * 16, 16)]
                z = a + bv + gv
                gate = 1.0 / (1.0 + jnp.exp(-z))
                mg_buf[e, pl.ds(g * 16, 16)] = m * gate
                mg_buf[e, pl.ds(64 + g * 16, 16)] = gate

    for p in range(2):
        b = 2 * p + c

        def fire_gidx(j, rowi, colgi, sem):
            base = ebase + j * K
            pltpu.make_async_copy(
                row4_hbm.at[pl.ds(b * E + base, K)], rowi, sem).start()
            pltpu.make_async_copy(
                col2_hbm.at[pl.ds(p * E + base, K)], colgi, sem).start()

        def drain_gidx(rowi, colgi, sem):
            pltpu.make_async_copy(row4_hbm.at[pl.ds(0, K)], rowi, sem).wait()
            pltpu.make_async_copy(col2_hbm.at[pl.ds(0, K)], colgi, sem).wait()

        def fire_sidx(j, colsi, sem):
            base = ebase + j * K
            pltpu.make_async_copy(
                col_hbm.at[pl.ds(base, K)], colsi, sem).start()

        def drain_sidx(colsi, sem):
            pltpu.make_async_copy(col_hbm.at[pl.ds(0, K)], colsi, sem).wait()

        def fire_gathers(j, rowi, colgi, am_buf, b_buf, g_buf):
            base = ebase + j * K
            pltpu.make_async_copy(am_hbm.at[rowi], am_buf, sem_g).start()
            pltpu.make_async_copy(b2_hbm.at[colgi], b_buf, sem_g).start()
            pltpu.make_async_copy(
                g2_hbm.at[pl.ds(p * E + base, K)], g_buf, sem_g).start()

        def drain_gathers(rowi, colgi, am_buf, b_buf, g_buf):
            pltpu.make_async_copy(am_hbm.at[rowi], am_buf, sem_g).wait()
            pltpu.make_async_copy(b2_hbm.at[colgi], b_buf, sem_g).wait()
            pltpu.make_async_copy(
                g2_hbm.at[pl.ds(0, K)], g_buf, sem_g).wait()

        def fire_scatter(mg_buf, colsi):
            pltpu.make_async_copy(
                mg_buf, acc_sp.at[colsi], sem_s).start(add=True)

        def drain_scatter(mg_buf, colsi):
            pltpu.make_async_copy(
                mg_buf, acc_sp.at[colsi], sem_s).wait()

        def zero_row(r, _):
            for g in range(8):
                am_a[r, pl.ds(g * 16, 16)] = jnp.zeros((16,), jnp.float32)
            return 0

        lax.fori_loop(0, RK, zero_row, 0)
        for q in range(RQ):
            pltpu.sync_copy(am_a.at[pl.ds(0, RK)],
                            acc_sp.at[pl.ds(rbase + q * RK, RK)])
        plsc.subcore_barrier()

        fire_gidx(0, rowi_a, colgi_a, sem_ia)
        fire_sidx(0, colsi_a, sem_sa)
        drain_gidx(rowi_a, colgi_a, sem_ia)
        fire_gathers(0, rowi_a, colgi_a, am_a, b_a, g_a)
        fire_gidx(1, rowi_b, colgi_b, sem_ib)
        fire_sidx(1, colsi_b, sem_sb)

        def pair_body(i2, _):
            j0 = 2 * i2

            drain_gathers(rowi_a, colgi_a, am_a, b_a, g_a)

            @pl.when(j0 > 0)
            def _():
                drain_scatter(mg_b, colsi_b)
                fire_sidx(j0 + 1, colsi_b, sem_sb)

            drain_gidx(rowi_b, colgi_b, sem_ib)
            fire_gathers(j0 + 1, rowi_b, colgi_b, am_b, b_b, g_b)

            @pl.when(j0 + 2 < NCHUNK)
            def _():
                fire_gidx(j0 + 2, rowi_a, colgi_a, sem_ia)

            drain_sidx(colsi_a, sem_sa)
            compute_edges(am_a, b_a, g_a, mg_a, K)
            fire_scatter(mg_a, colsi_a)

            drain_gathers(rowi_b, colgi_b, am_b, b_b, g_b)
            drain_scatter(mg_a, colsi_a)

            @pl.when(j0 + 2 < NCHUNK)
            def _():
                fire_sidx(j0 + 2, colsi_a, sem_sa)
                drain_gidx(rowi_a, colgi_a, sem_ia)
                fire_gathers(j0 + 2, rowi_a, colgi_a, am_a, b_a, g_a)

            @pl.when(j0 + 3 < NCHUNK)
            def _():
                fire_gidx(j0 + 3, rowi_b, colgi_b, sem_ib)

            drain_sidx(colsi_b, sem_sb)
            compute_edges(am_b, b_b, g_b, mg_b, K)
            fire_scatter(mg_b, colsi_b)
            return 0

        lax.fori_loop(0, NCHUNK // 2, pair_body, 0)
        drain_scatter(mg_b, colsi_b)

        plsc.subcore_barrier()

        for q in range(RQ):
            pltpu.sync_copy(acc_sp.at[pl.ds(rbase + q * RK, RK)],
                            am_a.at[pl.ds(0, RK)])
            pltpu.sync_copy(am_a.at[pl.ds(0, RK)],
                            acc_hbm.at[pl.ds(b * NP + rbase + q * RK, RK)])


def kernel(x, edge_index, edge_attr, W_root, b_root, W_msg, W_src_gate,
           W_dst_gate, W_edge_gate):
    f32 = jnp.float32
    i32 = jnp.int32

    row = edge_index[0].astype(i32)
    col = edge_index[1].astype(i32)
    row4 = (row[None, :] + (jnp.arange(4, dtype=i32) * N)[:, None]).reshape(-1)
    col2 = (col[None, :] + (jnp.arange(2, dtype=i32) * N)[:, None]).reshape(-1)

    w_am = jnp.concatenate(
        [W_src_gate.reshape(D, 4, 64).transpose(1, 0, 2),
         W_msg.reshape(D, 4, 64).transpose(1, 0, 2)], axis=-1)
    w_b2 = W_dst_gate.reshape(D, 2, 128).transpose(1, 0, 2)
    w_root2 = W_root.reshape(D, 2, 128).transpose(1, 0, 2)
    w_br = jnp.concatenate([w_b2, w_root2], axis=0)
    bias = jnp.concatenate(
        [jnp.zeros((2, 1, 128), f32),
         b_root.reshape(2, 1, 128)], axis=0)
    w_e2 = W_edge_gate.reshape(16, 2, 128).transpose(1, 0, 2)

    am, br = pl.pallas_call(
        _node_kernel,
        grid=(4, N // NB),
        in_specs=[
            pl.BlockSpec((NB, D), lambda b, i: (i, 0)),
            pl.BlockSpec((1, D, 128), lambda b, i: (b, 0, 0)),
            pl.BlockSpec((1, D, 128), lambda b, i: (b, 0, 0)),
            pl.BlockSpec((1, 1, 128), lambda b, i: (b, 0, 0)),
        ],
        out_specs=[
            pl.BlockSpec((1, NB, 128), lambda b, i: (b, i, 0)),
            pl.BlockSpec((1, NB, 128), lambda b, i: (b, i, 0)),
        ],
        out_shape=[
            jax.ShapeDtypeStruct((4, N, 128), f32),
            jax.ShapeDtypeStruct((4, N, 128), f32),
        ],
    )(x, w_am, w_br, bias)

    g2 = pl.pallas_call(
        _block_mm,
        grid=(2, E // EB),
        in_specs=[
            pl.BlockSpec((EB, 16), lambda p, i: (i, 0)),
            pl.BlockSpec((1, 16, 128), lambda p, i: (p, 0, 0)),
        ],
        out_specs=pl.BlockSpec((1, EB, 128), lambda p, i: (p, i, 0)),
        out_shape=jax.ShapeDtypeStruct((2, E, 128), f32),
    )(edge_attr, w_e2)

    sc_fn = functools.partial(
        pl.kernel,
        mesh=plsc.VectorSubcoreMesh(core_axis_name="c", subcore_axis_name="s"),
        out_type=jax.ShapeDtypeStruct((4 * NP, 128), f32),
        scratch_types=[
            pltpu.VMEM_SHARED((NP, 128), f32),
            pltpu.VMEM((K,), i32),
            pltpu.VMEM((K,), i32),
            pltpu.VMEM((K,), i32),
            pltpu.VMEM((K,), i32),
            pltpu.VMEM((K,), i32),
            pltpu.VMEM((K,), i32),
            pltpu.VMEM((K, 128), f32),
            pltpu.VMEM((K, 128), f32),
            pltpu.VMEM((K, 128), f32),
            pltpu.VMEM((K, 128), f32),
            pltpu.VMEM((K, 128), f32),
            pltpu.VMEM((K, 128), f32),
            pltpu.VMEM((K, 128), f32),
            pltpu.VMEM((K, 128), f32),
            pltpu.SemaphoreType.DMA,
            pltpu.SemaphoreType.DMA,
            pltpu.SemaphoreType.DMA,
            pltpu.SemaphoreType.DMA,
            pltpu.SemaphoreType.DMA,
            pltpu.SemaphoreType.DMA,
        ],
    )(_sc_edge_kernel)

    acc = sc_fn(am.reshape(4 * N, 128), br.reshape(4 * N, 128),
                g2.reshape(2 * E, 128), row4, col2, col)

    out = pl.pallas_call(
        _final_kernel,
        grid=(N // NB, 2),
        in_specs=[
            pl.BlockSpec((2, NB, 128), lambda i, j: (j, i, 0)),
            pl.BlockSpec((1, NB, 128), lambda i, j: (2 + j, i, 0)),
        ],
        out_specs=pl.BlockSpec((NB, 128), lambda i, j: (i, j)),
        out_shape=jax.ShapeDtypeStruct((N, D), f32),
    )(acc.reshape(4, NP, 128), br)

    return out


# speedup vs baseline: 1.0541x; 1.0541x over previous
import functools

import jax
import jax.numpy as jnp
from jax import lax
from jax.experimental import pallas as pl
from jax.experimental.pallas import tpu as pltpu
from jax.experimental.pallas import tpu_sc as plsc

N = 10000
E = 160000
D = 256

NP = 10240
NB = 1000
EB = 2000
K = 48
EPT = E // 16
NCHUNK = EPT // K
KTAIL = EPT - NCHUNK * K
RPT = NP // 16
RQ = 16
RK = RPT // RQ


def _block_mm(x_ref, w_ref, o_ref):
    o_ref[0] = jnp.dot(x_ref[...], w_ref[0], preferred_element_type=jnp.float32)


def _node_kernel(x_ref, wam_ref, wbr_ref, bias_ref, am_ref, br_ref):
    xb = x_ref[...]
    am_ref[0] = jnp.dot(xb, wam_ref[0], preferred_element_type=jnp.float32)
    br = jnp.dot(xb, wbr_ref[0], preferred_element_type=jnp.float32) + bias_ref[0]
    j = pl.program_id(0)

    @pl.when(j == 2)
    def _():
        br_ref[0] = br + xb[:, :128]

    @pl.when(j == 3)
    def _():
        br_ref[0] = br + xb[:, 128:]

    @pl.when(j < 2)
    def _():
        br_ref[0] = br


def _final_kernel(acc_ref, r_ref, o_ref):
    a0 = acc_ref[0]
    a1 = acc_ref[1]
    msg = jnp.concatenate([a0[:, :64], a1[:, :64]], axis=1)
    norm = jnp.concatenate([a0[:, 64:], a1[:, 64:]], axis=1)
    o_ref[...] = msg / jnp.maximum(norm, 1e-6) + r_ref[0]


def _sc_edge_kernel(am_hbm, b2_hbm, g2_hbm, row4_hbm, col2_hbm, col_hbm,
                    acc_hbm, acc_sp, rowi_a, colgi_a, colsi_a, rowi_b,
                    colgi_b, colsi_b, rowt, colgt, colst, am_a, b_a, g_a,
                    mg_a, am_b, b_b, g_b, mg_b, sem_ia, sem_ib, sem_sa,
                    sem_sb, sem_g, sem_s):
    c = lax.axis_index("c")
    s = lax.axis_index("s")
    coff = c * 64
    rbase = s * RPT
    ebase = s * EPT

    def compute_edges(am_buf, b_buf, g_buf, mg_buf, nedges):
        @plsc.parallel_loop(0, nedges, step=2, unroll=2)
        def edge_body(e0):
            r = lax.shift_right_logical(e0, 1)
            for h in range(2):
                e = e0 + h
                for g in range(4):
                    a = am_buf[e, pl.ds(g * 16, 16)]
                    m = am_buf[e, pl.ds(64 + g * 16, 16)]
                    bv = b_buf[e, pl.ds(coff + g * 16, 16)]
                    gv = g_buf[r, pl.ds(64 * h + g * 16, 16)]
                    z = a + bv + gv
                    gate = 1.0 / (1.0 + jnp.exp(-z))
                    mg_buf[e, pl.ds(g * 16, 16)] = m * gate
                    mg_buf[e, pl.ds(64 + g * 16, 16)] = gate

    for p in range(2):
        b = 2 * p + c

        def fire_gidx(j, rowi, colgi, sem):
            base = ebase + j * K
            pltpu.make_async_copy(
                row4_hbm.at[pl.ds(b * E + base, K)], rowi, sem).start()
            pltpu.make_async_copy(
                col2_hbm.at[pl.ds(p * E + base, K)], colgi, sem).start()

        def drain_gidx(rowi, colgi, sem):
            pltpu.make_async_copy(row4_hbm.at[pl.ds(0, K)], rowi, sem).wait()
            pltpu.make_async_copy(col2_hbm.at[pl.ds(0, K)], colgi, sem).wait()

        def fire_sidx(j, colsi, sem):
            base = ebase + j * K
            pltpu.make_async_copy(
                col_hbm.at[pl.ds(base, K)], colsi, sem).start()

        def drain_sidx(colsi, sem):
            pltpu.make_async_copy(col_hbm.at[pl.ds(0, K)], colsi, sem).wait()

        def fire_gathers(j, rowi, colgi, am_buf, b_buf, g_buf):
            base2 = s * (EPT // 2) + j * (K // 2)
            pltpu.make_async_copy(am_hbm.at[rowi], am_buf, sem_g).start()
            pltpu.make_async_copy(b2_hbm.at[colgi], b_buf, sem_g).start()
            pltpu.make_async_copy(
                g2_hbm.at[pl.ds(b * (E // 2) + base2, K // 2)],
                g_buf, sem_g).start()

        def drain_gathers(rowi, colgi, am_buf, b_buf, g_buf):
            pltpu.make_async_copy(am_hbm.at[rowi], am_buf, sem_g).wait()
            pltpu.make_async_copy(b2_hbm.at[colgi], b_buf, sem_g).wait()
            pltpu.make_async_copy(
                g2_hbm.at[pl.ds(0, K // 2)], g_buf, sem_g).wait()

        def fire_scatter(mg_buf, colsi):
            pltpu.make_async_copy(
                mg_buf, acc_sp.at[colsi], sem_s).start(add=True)

        def drain_scatter(mg_buf, colsi):
            pltpu.make_async_copy(
                mg_buf, acc_sp.at[colsi], sem_s).wait()

        def zero_row(r, _):
            for g in range(8):
                am_a[r, pl.ds(g * 16, 16)] = jnp.zeros((16,), jnp.float32)
            return 0

        lax.fori_loop(0, RK, zero_row, 0)
        for q in range(RQ):
            pltpu.sync_copy(am_a.at[pl.ds(0, RK)],
                            acc_sp.at[pl.ds(rbase + q * RK, RK)])
        plsc.subcore_barrier()

        fire_gidx(0, rowi_a, colgi_a, sem_ia)
        fire_sidx(0, colsi_a, sem_sa)
        drain_gidx(rowi_a, colgi_a, sem_ia)
        fire_gathers(0, rowi_a, colgi_a, am_a, b_a, g_a)
        fire_gidx(1, rowi_b, colgi_b, sem_ib)
        fire_sidx(1, colsi_b, sem_sb)

        def pair_body(i2, _):
            j0 = 2 * i2

            drain_gathers(rowi_a, colgi_a, am_a, b_a, g_a)

            @pl.when(j0 > 0)
            def _():
                drain_scatter(mg_b, colsi_b)
                fire_sidx(j0 + 1, colsi_b, sem_sb)

            drain_gidx(rowi_b, colgi_b, sem_ib)
            fire_gathers(j0 + 1, rowi_b, colgi_b, am_b, b_b, g_b)

            @pl.when(j0 + 2 < NCHUNK)
            def _():
                fire_gidx(j0 + 2, rowi_a, colgi_a, sem_ia)

            drain_sidx(colsi_a, sem_sa)
            compute_edges(am_a, b_a, g_a, mg_a, K)
            fire_scatter(mg_a, colsi_a)

            drain_gathers(rowi_b, colgi_b, am_b, b_b, g_b)
            drain_scatter(mg_a, colsi_a)

            @pl.when(j0 + 2 < NCHUNK)
            def _():
                fire_sidx(j0 + 2, colsi_a, sem_sa)
                drain_gidx(rowi_a, colgi_a, sem_ia)
                fire_gathers(j0 + 2, rowi_a, colgi_a, am_a, b_a, g_a)

            @pl.when(j0 + 3 < NCHUNK)
            def _():
                fire_gidx(j0 + 3, rowi_b, colgi_b, sem_ib)

            drain_sidx(colsi_b, sem_sb)
            compute_edges(am_b, b_b, g_b, mg_b, K)
            fire_scatter(mg_b, colsi_b)
            return 0

        lax.fori_loop(0, NCHUNK // 2, pair_body, 0)
        drain_scatter(mg_b, colsi_b)

        tbase = ebase + NCHUNK * K
        tbase2 = s * (EPT // 2) + NCHUNK * (K // 2)
        pltpu.sync_copy(row4_hbm.at[pl.ds(b * E + tbase, KTAIL)], rowt)
        pltpu.sync_copy(col2_hbm.at[pl.ds(p * E + tbase, KTAIL)], colgt)
        pltpu.sync_copy(col_hbm.at[pl.ds(tbase, KTAIL)], colst)
        pltpu.sync_copy(am_hbm.at[rowt], am_a.at[pl.ds(0, KTAIL)])
        pltpu.sync_copy(b2_hbm.at[colgt], b_a.at[pl.ds(0, KTAIL)])
        pltpu.sync_copy(g2_hbm.at[pl.ds(b * (E // 2) + tbase2, KTAIL // 2)],
                        g_a.at[pl.ds(0, KTAIL // 2)])
        compute_edges(am_a, b_a, g_a, mg_a, KTAIL)
        pltpu.sync_copy(mg_a.at[pl.ds(0, KTAIL)], acc_sp.at[colst], add=True)

        plsc.subcore_barrier()

        for q in range(RQ):
            pltpu.sync_copy(acc_sp.at[pl.ds(rbase + q * RK, RK)],
                            am_a.at[pl.ds(0, RK)])
            pltpu.sync_copy(am_a.at[pl.ds(0, RK)],
                            acc_hbm.at[pl.ds(b * NP + rbase + q * RK, RK)])


def kernel(x, edge_index, edge_attr, W_root, b_root, W_msg, W_src_gate,
           W_dst_gate, W_edge_gate):
    f32 = jnp.float32
    i32 = jnp.int32

    row = edge_index[0].astype(i32)
    col = edge_index[1].astype(i32)
    row4 = (row[None, :] + (jnp.arange(4, dtype=i32) * N)[:, None]).reshape(-1)
    col2 = (col[None, :] + (jnp.arange(2, dtype=i32) * N)[:, None]).reshape(-1)

    w_am = jnp.concatenate(
        [W_src_gate.reshape(D, 4, 64).transpose(1, 0, 2),
         W_msg.reshape(D, 4, 64).transpose(1, 0, 2)], axis=-1)
    w_b2 = W_dst_gate.reshape(D, 2, 128).transpose(1, 0, 2)
    w_root2 = W_root.reshape(D, 2, 128).transpose(1, 0, 2)
    w_br = jnp.concatenate([w_b2, w_root2], axis=0)
    bias = jnp.concatenate(
        [jnp.zeros((2, 1, 128), f32),
         b_root.reshape(2, 1, 128)], axis=0)
    w_e4 = W_edge_gate.reshape(16, 4, 64).transpose(1, 0, 2)
    w_gbd = jnp.zeros((4, 32, 128), f32)
    w_gbd = w_gbd.at[:, :16, :64].set(w_e4)
    w_gbd = w_gbd.at[:, 16:, 64:].set(w_e4)
    ea2 = edge_attr.reshape(E // 2, 32)

    am, br = pl.pallas_call(
        _node_kernel,
        grid=(4, N // NB),
        in_specs=[
            pl.BlockSpec((NB, D), lambda b, i: (i, 0)),
            pl.BlockSpec((1, D, 128), lambda b, i: (b, 0, 0)),
            pl.BlockSpec((1, D, 128), lambda b, i: (b, 0, 0)),
            pl.BlockSpec((1, 1, 128), lambda b, i: (b, 0, 0)),
        ],
        out_specs=[
            pl.BlockSpec((1, NB, 128), lambda b, i: (b, i, 0)),
            pl.BlockSpec((1, NB, 128), lambda b, i: (b, i, 0)),
        ],
        out_shape=[
            jax.ShapeDtypeStruct((4, N, 128), f32),
            jax.ShapeDtypeStruct((4, N, 128), f32),
        ],
    )(x, w_am, w_br, bias)

    g2 = pl.pallas_call(
        _block_mm,
        grid=(4, (E // 2) // EB),
        in_specs=[
            pl.BlockSpec((EB, 32), lambda bb, i: (i, 0)),
            pl.BlockSpec((1, 32, 128), lambda bb, i: (bb, 0, 0)),
        ],
        out_specs=pl.BlockSpec((1, EB, 128), lambda bb, i: (bb, i, 0)),
        out_shape=jax.ShapeDtypeStruct((4, E // 2, 128), f32),
    )(ea2, w_gbd)

    sc_fn = functools.partial(
        pl.kernel,
        mesh=plsc.VectorSubcoreMesh(core_axis_name="c", subcore_axis_name="s"),
        out_type=jax.ShapeDtypeStruct((4 * NP, 128), f32),
        scratch_types=[
            pltpu.VMEM_SHARED((NP, 128), f32),
            pltpu.VMEM((K,), i32),
            pltpu.VMEM((K,), i32),
            pltpu.VMEM((K,), i32),
            pltpu.VMEM((K,), i32),
            pltpu.VMEM((K,), i32),
            pltpu.VMEM((K,), i32),
            pltpu.VMEM((KTAIL,), i32),
            pltpu.VMEM((KTAIL,), i32),
            pltpu.VMEM((KTAIL,), i32),
            pltpu.VMEM((K, 128), f32),
            pltpu.VMEM((K, 128), f32),
            pltpu.VMEM((K // 2, 128), f32),
            pltpu.VMEM((K, 128), f32),
            pltpu.VMEM((K, 128), f32),
            pltpu.VMEM((K, 128), f32),
            pltpu.VMEM((K // 2, 128), f32),
            pltpu.VMEM((K, 128), f32),
            pltpu.SemaphoreType.DMA,
            pltpu.SemaphoreType.DMA,
            pltpu.SemaphoreType.DMA,
            pltpu.SemaphoreType.DMA,
            pltpu.SemaphoreType.DMA,
            pltpu.SemaphoreType.DMA,
        ],
    )(_sc_edge_kernel)

    acc = sc_fn(am.reshape(4 * N, 128), br.reshape(4 * N, 128),
                g2.reshape(4 * (E // 2), 128), row4, col2, col)

    out = pl.pallas_call(
        _final_kernel,
        grid=(N // NB, 2),
        in_specs=[
            pl.BlockSpec((2, NB, 128), lambda i, j: (j, i, 0)),
            pl.BlockSpec((1, NB, 128), lambda i, j: (2 + j, i, 0)),
        ],
        out_specs=pl.BlockSpec((NB, 128), lambda i, j: (i, j)),
        out_shape=jax.ShapeDtypeStruct((N, D), f32),
    )(acc.reshape(4, NP, 128), br)

    return out


# speedup vs baseline: 1.0900x; 1.0341x over previous
import functools

import jax
import jax.numpy as jnp
from jax import lax
from jax.experimental import pallas as pl
from jax.experimental.pallas import tpu as pltpu
from jax.experimental.pallas import tpu_sc as plsc

N = 10000
E = 160000
D = 256

NP = 10240
NB = 1000
EB = 2000
K = 48
EPT = E // 16
NCHUNK = EPT // K
KTAIL = EPT - NCHUNK * K
RPT = NP // 16
RQ = 16
RK = RPT // RQ


def _block_mm(x_ref, w_ref, o_ref):
    o_ref[0] = jnp.dot(x_ref[...], w_ref[0], preferred_element_type=jnp.float32)


def _node_kernel(x_ref, wam_ref, wbr_ref, bias_ref, am_ref, br_ref):
    xb = x_ref[...]
    am_ref[0] = jnp.dot(xb, wam_ref[0], preferred_element_type=jnp.float32)
    br = jnp.dot(xb, wbr_ref[0], preferred_element_type=jnp.float32) + bias_ref[0]
    j = pl.program_id(0)

    @pl.when(j == 2)
    def _():
        br_ref[0] = br + xb[:, :128]

    @pl.when(j == 3)
    def _():
        br_ref[0] = br + xb[:, 128:]

    @pl.when(j < 2)
    def _():
        br_ref[0] = br


def _final_kernel(acc_ref, r_ref, o_ref):
    a0 = acc_ref[0]
    a1 = acc_ref[1]
    msg = jnp.concatenate([a0[:, :64], a1[:, :64]], axis=1)
    norm = jnp.concatenate([a0[:, 64:], a1[:, 64:]], axis=1)
    o_ref[...] = msg / jnp.maximum(norm, 1e-6) + r_ref[0]


def _sc_edge_kernel(am_hbm, b2_hbm, g2_hbm, row4_hbm, col2_hbm, col_hbm,
                    acc_hbm, acc_sp, rowi_a, colgi_a, colsi_a, rowi_b,
                    colgi_b, colsi_b, rowt, colgt, colst, am_a, b_a, g_a,
                    mg_a, am_b, b_b, g_b, mg_b, sem_ia, sem_ib, sem_sa,
                    sem_sb, sem_g, sem_s):
    c = lax.axis_index("c")
    s = lax.axis_index("s")
    coff = c * 64
    rbase = s * RPT
    ebase = s * EPT

    def compute_edges(am_buf, b_buf, g_buf, mg_buf, nedges):
        @plsc.parallel_loop(0, nedges, step=2, unroll=2)
        def edge_body(e0):
            r = lax.shift_right_logical(e0, 1)
            for h in range(2):
                e = e0 + h
                for g in range(4):
                    a = am_buf[e, pl.ds(g * 16, 16)]
                    m = am_buf[e, pl.ds(64 + g * 16, 16)]
                    bv = b_buf[e, pl.ds(coff + g * 16, 16)]
                    gv = g_buf[r, pl.ds(64 * h + g * 16, 16)]
                    z = a + bv + gv
                    gate = 1.0 / (1.0 + jnp.exp(-z))
                    mg_buf[e, pl.ds(g * 16, 16)] = m * gate
                    mg_buf[e, pl.ds(64 + g * 16, 16)] = gate

    for p in range(2):
        b = 2 * p + c

        def fire_gidx(j, rowi, colgi, sem):
            base = ebase + j * K
            pltpu.make_async_copy(
                row4_hbm.at[pl.ds(b * E + base, K)], rowi, sem).start()
            pltpu.make_async_copy(
                col2_hbm.at[pl.ds(p * E + base, K)], colgi, sem).start()

        def drain_gidx(rowi, colgi, sem):
            pltpu.make_async_copy(row4_hbm.at[pl.ds(0, K)], rowi, sem).wait()
            pltpu.make_async_copy(col2_hbm.at[pl.ds(0, K)], colgi, sem).wait()

        def fire_sidx(j, colsi, sem):
            base = ebase + j * K
            pltpu.make_async_copy(
                col_hbm.at[pl.ds(base, K)], colsi, sem).start()

        def drain_sidx(colsi, sem):
            pltpu.make_async_copy(col_hbm.at[pl.ds(0, K)], colsi, sem).wait()

        def fire_gathers(j, rowi, colgi, am_buf, b_buf, g_buf):
            base2 = s * (EPT // 2) + j * (K // 2)
            pltpu.make_async_copy(am_hbm.at[rowi], am_buf, sem_g).start()
            pltpu.make_async_copy(b2_hbm.at[colgi], b_buf, sem_g).start()
            pltpu.make_async_copy(
                g2_hbm.at[pl.ds(b * (E // 2) + base2, K // 2)],
                g_buf, sem_g).start()

        def drain_gathers(rowi, colgi, am_buf, b_buf, g_buf):
            pltpu.make_async_copy(am_hbm.at[rowi], am_buf, sem_g).wait()
            pltpu.make_async_copy(b2_hbm.at[colgi], b_buf, sem_g).wait()
            pltpu.make_async_copy(
                g2_hbm.at[pl.ds(0, K // 2)], g_buf, sem_g).wait()

        def fire_scatter(mg_buf, colsi):
            pltpu.make_async_copy(
                mg_buf, acc_sp.at[colsi], sem_s).start(add=True)

        def drain_scatter(mg_buf, colsi):
            pltpu.make_async_copy(
                mg_buf, acc_sp.at[colsi], sem_s).wait()

        def zero_row(r, _):
            for g in range(8):
                am_a[r, pl.ds(g * 16, 16)] = jnp.zeros((16,), jnp.float32)
            return 0

        lax.fori_loop(0, RK, zero_row, 0)
        for q in range(RQ):
            pltpu.make_async_copy(am_a.at[pl.ds(0, RK)],
                                  acc_sp.at[pl.ds(rbase + q * RK, RK)],
                                  sem_g).start()
        for q in range(RQ):
            pltpu.make_async_copy(am_a.at[pl.ds(0, RK)],
                                  acc_sp.at[pl.ds(rbase + q * RK, RK)],
                                  sem_g).wait()
        plsc.subcore_barrier()

        fire_gidx(0, rowi_a, colgi_a, sem_ia)
        fire_sidx(0, colsi_a, sem_sa)
        drain_gidx(rowi_a, colgi_a, sem_ia)
        fire_gathers(0, rowi_a, colgi_a, am_a, b_a, g_a)
        fire_gidx(1, rowi_b, colgi_b, sem_ib)
        fire_sidx(1, colsi_b, sem_sb)

        def pair_body(i2, _):
            j0 = 2 * i2

            drain_gathers(rowi_a, colgi_a, am_a, b_a, g_a)

            @pl.when(j0 > 0)
            def _():
                drain_scatter(mg_b, colsi_b)
                fire_sidx(j0 + 1, colsi_b, sem_sb)

            drain_gidx(rowi_b, colgi_b, sem_ib)
            fire_gathers(j0 + 1, rowi_b, colgi_b, am_b, b_b, g_b)

            @pl.when(j0 + 2 < NCHUNK)
            def _():
                fire_gidx(j0 + 2, rowi_a, colgi_a, sem_ia)

            drain_sidx(colsi_a, sem_sa)
            compute_edges(am_a, b_a, g_a, mg_a, K)
            fire_scatter(mg_a, colsi_a)

            drain_gathers(rowi_b, colgi_b, am_b, b_b, g_b)
            drain_scatter(mg_a, colsi_a)

            @pl.when(j0 + 2 < NCHUNK)
            def _():
                fire_sidx(j0 + 2, colsi_a, sem_sa)
                drain_gidx(rowi_a, colgi_a, sem_ia)
                fire_gathers(j0 + 2, rowi_a, colgi_a, am_a, b_a, g_a)

            @pl.when(j0 + 3 < NCHUNK)
            def _():
                fire_gidx(j0 + 3, rowi_b, colgi_b, sem_ib)

            drain_sidx(colsi_b, sem_sb)
            compute_edges(am_b, b_b, g_b, mg_b, K)
            fire_scatter(mg_b, colsi_b)
            return 0

        lax.fori_loop(0, NCHUNK // 2, pair_body, 0)
        drain_scatter(mg_b, colsi_b)

        tbase = ebase + NCHUNK * K
        tbase2 = s * (EPT // 2) + NCHUNK * (K // 2)
        pltpu.sync_copy(row4_hbm.at[pl.ds(b * E + tbase, KTAIL)], rowt)
        pltpu.sync_copy(col2_hbm.at[pl.ds(p * E + tbase, KTAIL)], colgt)
        pltpu.sync_copy(col_hbm.at[pl.ds(tbase, KTAIL)], colst)
        pltpu.sync_copy(am_hbm.at[rowt], am_a.at[pl.ds(0, KTAIL)])
        pltpu.sync_copy(b2_hbm.at[colgt], b_a.at[pl.ds(0, KTAIL)])
        pltpu.sync_copy(g2_hbm.at[pl.ds(b * (E // 2) + tbase2, KTAIL // 2)],
                        g_a.at[pl.ds(0, KTAIL // 2)])
        compute_edges(am_a, b_a, g_a, mg_a, KTAIL)
        pltpu.sync_copy(mg_a.at[pl.ds(0, KTAIL)], acc_sp.at[colst], add=True)

        plsc.subcore_barrier()

        def wb_load(q, buf):
            pltpu.make_async_copy(acc_sp.at[pl.ds(rbase + q * RK, RK)],
                                  buf.at[pl.ds(0, RK)], sem_g).start()

        def wb_load_wait(q, buf):
            pltpu.make_async_copy(acc_sp.at[pl.ds(rbase + q * RK, RK)],
                                  buf.at[pl.ds(0, RK)], sem_g).wait()

        def wb_store(q, buf):
            pltpu.make_async_copy(buf.at[pl.ds(0, RK)],
                                  acc_hbm.at[pl.ds(b * NP + rbase + q * RK, RK)],
                                  sem_s).start()

        def wb_store_wait(q, buf):
            pltpu.make_async_copy(buf.at[pl.ds(0, RK)],
                                  acc_hbm.at[pl.ds(b * NP + rbase + q * RK, RK)],
                                  sem_s).wait()

        bufs = (am_a, am_b, mg_a, mg_b)
        for q in range(4):
            wb_load(q, bufs[q])
        for q in range(RQ):
            wb_load_wait(q, bufs[q % 4])
            wb_store(q, bufs[q % 4])
            qq = q - 2
            if qq >= 0 and qq + 4 < RQ:
                wb_store_wait(qq, bufs[qq % 4])
                wb_load(qq + 4, bufs[qq % 4])
        for q in range(RQ - 4, RQ):
            wb_store_wait(q, bufs[q % 4])


def kernel(x, edge_index, edge_attr, W_root, b_root, W_msg, W_src_gate,
           W_dst_gate, W_edge_gate):
    f32 = jnp.float32
    i32 = jnp.int32

    row = edge_index[0].astype(i32)
    col = edge_index[1].astype(i32)
    row4 = (row[None, :] + (jnp.arange(4, dtype=i32) * N)[:, None]).reshape(-1)
    col2 = (col[None, :] + (jnp.arange(2, dtype=i32) * N)[:, None]).reshape(-1)

    w_am = jnp.concatenate(
        [W_src_gate.reshape(D, 4, 64).transpose(1, 0, 2),
         W_msg.reshape(D, 4, 64).transpose(1, 0, 2)], axis=-1)
    w_b2 = W_dst_gate.reshape(D, 2, 128).transpose(1, 0, 2)
    w_root2 = W_root.reshape(D, 2, 128).transpose(1, 0, 2)
    w_br = jnp.concatenate([w_b2, w_root2], axis=0)
    bias = jnp.concatenate(
        [jnp.zeros((2, 1, 128), f32),
         b_root.reshape(2, 1, 128)], axis=0)
    w_e4 = W_edge_gate.reshape(16, 4, 64).transpose(1, 0, 2)
    w_gbd = jnp.zeros((4, 32, 128), f32)
    w_gbd = w_gbd.at[:, :16, :64].set(w_e4)
    w_gbd = w_gbd.at[:, 16:, 64:].set(w_e4)
    ea2 = edge_attr.reshape(E // 2, 32)

    am, br = pl.pallas_call(
        _node_kernel,
        grid=(4, N // NB),
        in_specs=[
            pl.BlockSpec((NB, D), lambda b, i: (i, 0)),
            pl.BlockSpec((1, D, 128), lambda b, i: (b, 0, 0)),
            pl.BlockSpec((1, D, 128), lambda b, i: (b, 0, 0)),
            pl.BlockSpec((1, 1, 128), lambda b, i: (b, 0, 0)),
        ],
        out_specs=[
            pl.BlockSpec((1, NB, 128), lambda b, i: (b, i, 0)),
            pl.BlockSpec((1, NB, 128), lambda b, i: (b, i, 0)),
        ],
        out_shape=[
            jax.ShapeDtypeStruct((4, N, 128), f32),
            jax.ShapeDtypeStruct((4, N, 128), f32),
        ],
    )(x, w_am, w_br, bias)

    g2 = pl.pallas_call(
        _block_mm,
        grid=(4, (E // 2) // EB),
        in_specs=[
            pl.BlockSpec((EB, 32), lambda bb, i: (i, 0)),
            pl.BlockSpec((1, 32, 128), lambda bb, i: (bb, 0, 0)),
        ],
        out_specs=pl.BlockSpec((1, EB, 128), lambda bb, i: (bb, i, 0)),
        out_shape=jax.ShapeDtypeStruct((4, E // 2, 128), f32),
    )(ea2, w_gbd)

    sc_fn = functools.partial(
        pl.kernel,
        mesh=plsc.VectorSubcoreMesh(core_axis_name="c", subcore_axis_name="s"),
        out_type=jax.ShapeDtypeStruct((4 * NP, 128), f32),
        scratch_types=[
            pltpu.VMEM_SHARED((NP, 128), f32),
            pltpu.VMEM((K,), i32),
            pltpu.VMEM((K,), i32),
            pltpu.VMEM((K,), i32),
            pltpu.VMEM((K,), i32),
            pltpu.VMEM((K,), i32),
            pltpu.VMEM((K,), i32),
            pltpu.VMEM((KTAIL,), i32),
            pltpu.VMEM((KTAIL,), i32),
            pltpu.VMEM((KTAIL,), i32),
            pltpu.VMEM((K, 128), f32),
            pltpu.VMEM((K, 128), f32),
            pltpu.VMEM((K // 2, 128), f32),
            pltpu.VMEM((K, 128), f32),
            pltpu.VMEM((K, 128), f32),
            pltpu.VMEM((K, 128), f32),
            pltpu.VMEM((K // 2, 128), f32),
            pltpu.VMEM((K, 128), f32),
            pltpu.SemaphoreType.DMA,
            pltpu.SemaphoreType.DMA,
            pltpu.SemaphoreType.DMA,
            pltpu.SemaphoreType.DMA,
            pltpu.SemaphoreType.DMA,
            pltpu.SemaphoreType.DMA,
        ],
    )(_sc_edge_kernel)

    acc = sc_fn(am.reshape(4 * N, 128), br.reshape(4 * N, 128),
                g2.reshape(4 * (E // 2), 128), row4, col2, col)

    out = pl.pallas_call(
        _final_kernel,
        grid=(N // NB, 2),
        in_specs=[
            pl.BlockSpec((2, NB, 128), lambda i, j: (j, i, 0)),
            pl.BlockSpec((1, NB, 128), lambda i, j: (2 + j, i, 0)),
        ],
        out_specs=pl.BlockSpec((NB, 128), lambda i, j: (i, j)),
        out_shape=jax.ShapeDtypeStruct((N, D), f32),
    )(acc.reshape(4, NP, 128), br)

    return out
